# baseline (speedup 1.0000x reference)
import jax
import jax.numpy as jnp
from jax import lax
from jax.experimental import pallas as pl
from jax.experimental.pallas import tpu as pltpu
from jax.experimental.pallas import tpu_sc as plsc

N_NODES = 10000
D = 256
N_EDGES = 160000

NC = 2
NS = 16
NF = 2
FW = D // NF
NR = NS // NF
HALF = N_NODES // NC
HP = 5120
Q = HALF // 2
QP = 2560
RH = QP // NR
TPH = HP // NS
ECH = N_EDGES // NS
SUB = 2000
NSUB = ECH // SUB
CAP = ECH + 240
TRASH = CAP - 16
WN = 512
GB = 128
NEG = 0.2
PKM = 16384


def _tc_body(x_ref, wt_ref, as_ref, ad_ref, h_ref, asc_ref, adt_ref):
    h = jnp.dot(x_ref[...], wt_ref[...], preferred_element_type=jnp.float32)
    h_ref[...] = h
    asc_ref[...] = jnp.dot(h, as_ref[...], preferred_element_type=jnp.float32)
    adt_ref[...] = jnp.dot(h, ad_ref[...], preferred_element_type=jnp.float32)


@jax.jit
def _tc_transform(x, wt, att_s, att_d):
    blk = 1000
    grid = N_NODES // blk
    return pl.pallas_call(
        _tc_body,
        grid=(grid,),
        in_specs=[
            pl.BlockSpec((blk, D), lambda i: (i, 0)),
            pl.BlockSpec((D, D), lambda i: (0, 0)),
            pl.BlockSpec((D, 1), lambda i: (0, 0)),
            pl.BlockSpec((D, 1), lambda i: (0, 0)),
        ],
        out_specs=[
            pl.BlockSpec((blk, D), lambda i: (i, 0)),
            pl.BlockSpec((blk, 1), lambda i: (i, 0)),
            pl.BlockSpec((blk, 1), lambda i: (i, 0)),
        ],
        out_shape=[
            jax.ShapeDtypeStruct((N_NODES, D), jnp.float32),
            jax.ShapeDtypeStruct((N_NODES, 1), jnp.float32),
            jax.ShapeDtypeStruct((N_NODES, 1), jnp.float32),
        ],
    )(x, wt, att_s, att_d)


def _zero_1d(ref, n16, dtype):
    z = jnp.zeros((16,), dtype)

    def b(i, _):
        ref[pl.ds(i * 16, 16)] = z
        return 0

    lax.fori_loop(0, n16, b, 0)


def _prefix_incl(mi, lane):
    ps = mi
    for sh in (1, 2, 4, 8):
        pidx = jnp.maximum(lane - sh, 0)
        sh_v = ps.at[pidx].get(mode="promise_in_bounds")
        ps = ps + jnp.where(lane >= sh, sh_v, 0)
    return ps


def _sc_body(h3_hbm, asrc_hbm, adst_hbm, src_hbm, tgt_hbm,
             out_hbm, pkl_hbm, cfl_hbm,
             asrc_v, adh_v, m_v, s_v, srcs_v, tgts_v,
             pk_v, cf_v, agg_v, cnts_v, c16_v,
             wpk_v, wcf_v, lpk_v, lcf_v, widx_v, rows_v, acc_v, tmp_v,
             counts_sh, parts_sh, ssum_sh, sem):
    c = lax.axis_index("c")
    s = lax.axis_index("s")
    lo = c * HALF
    f = s % NF
    n = s // NF
    lane = lax.iota(jnp.int32, 16)

    pltpu.sync_copy(asrc_hbm, asrc_v)
    pltpu.sync_copy(adst_hbm.at[pl.ds(lo, HALF)], adh_v.at[pl.ds(0, HALF)])
    _zero_1d(adh_v.at[pl.ds(HALF, HP - HALF)], (HP - HALF) // 16, jnp.float32)
    _zero_1d(s_v, HP // 16, jnp.float32)

    def amax_b(i, acc):
        return jnp.maximum(acc, asrc_v[pl.ds(i * 16, 16)])

    mx = lax.fori_loop(0, N_NODES // 16, amax_b,
                       jnp.full((16,), -jnp.inf, jnp.float32))
    for sh in (8, 4, 2, 1):
        perm = jnp.bitwise_xor(lane, sh)
        mx = jnp.maximum(mx, mx.at[perm].get(mode="promise_in_bounds"))
    astar = mx

    def m_b(i, _):
        z = astar + adh_v[pl.ds(i * 16, 16)]
        lr = jnp.maximum(z, NEG * z)
        m_v[pl.ds(i * 16, 16)] = jnp.maximum(lr, 0.0)
        return 0

    lax.fori_loop(0, HP // 16, m_b, 0)

    def scan_b(k, _):
        pltpu.sync_copy(src_hbm.at[pl.ds(s * ECH + k * SUB, SUB)], srcs_v)
        pltpu.sync_copy(tgt_hbm.at[pl.ds(s * ECH + k * SUB, SUB)], tgts_v)

        def pa_b(g, _):
            s16 = srcs_v[pl.ds(g * 16, 16)]
            t16 = tgts_v[pl.ds(g * 16, 16)]
            l16 = t16 - lo
            inh = (l16 >= 0) & (l16 < HALF)
            lc = jnp.clip(l16, 0, HALF - 1)
            z = (plsc.load_gather(asrc_v, [s16])
                 + plsc.load_gather(adh_v, [lc]))
            al = jnp.maximum(z, NEG * z)
            e = jnp.exp(al - plsc.load_gather(m_v, [lc]))
            e = jnp.where(inh, e, 0.0)
            plsc.addupdate_scatter(s_v, [lc], e)
            return 0

        lax.fori_loop(0, SUB // 16, pa_b, 0)
        return 0

    lax.fori_loop(0, NSUB, scan_b, 0)

    pltpu.sync_copy(s_v, parts_sh.at[pl.ds(s * HP, HP)])
    plsc.subcore_barrier()
    _zero_1d(acc_v, TPH // 16, jnp.float32)

    def red_b(k, _):
        pltpu.sync_copy(parts_sh.at[pl.ds(k * HP + s * TPH, TPH)], tmp_v)

        def add_b(i, _):
            a = acc_v[pl.ds(i * 16, 16)]
            acc_v[pl.ds(i * 16, 16)] = a + tmp_v[pl.ds(i * 16, 16)]
            return 0

        lax.fori_loop(0, TPH // 16, add_b, 0)
        return 0

    lax.fori_loop(0, NS, red_b, 0)
    pltpu.sync_copy(acc_v, ssum_sh.at[pl.ds(s * TPH, TPH)])
    plsc.subcore_barrier()
    pltpu.sync_copy(ssum_sh, s_v)

    for q in (0, 1):
        _zero_1d(pk_v, CAP // 16, jnp.int32)
        _zero_1d(cf_v, CAP // 16, jnp.float32)

        def csc_b(k, cntv):
            pltpu.sync_copy(src_hbm.at[pl.ds(s * ECH + k * SUB, SUB)], srcs_v)
            pltpu.sync_copy(tgt_hbm.at[pl.ds(s * ECH + k * SUB, SUB)], tgts_v)

            def comp_b(g, cv):
                s16 = srcs_v[pl.ds(g * 16, 16)]
                t16 = tgts_v[pl.ds(g * 16, 16)]
                l16 = t16 - lo
                mq = (l16 >= q * Q) & (l16 < (q + 1) * Q)
                lc = jnp.clip(l16, 0, HALF - 1)
                z = (plsc.load_gather(asrc_v, [s16])
                     + plsc.load_gather(adh_v, [lc]))
                al = jnp.maximum(z, NEG * z)
                e = jnp.exp(al - plsc.load_gather(m_v, [lc]))
                ssum = plsc.load_gather(s_v, [lc])
                cf = e / (ssum + 1e-16)
                tq = jnp.clip(l16 - q * Q, 0, QP - 1)
                pk = tq * PKM + s16
                mi = mq.astype(jnp.int32)
                ps = _prefix_incl(mi, lane)
                dest = jnp.where(mq, cv + ps - mi, TRASH + lane)
                plsc.store_scatter(pk_v, [dest], pk)
                plsc.store_scatter(cf_v, [dest], cf)
                return cv + ps[15]

            return lax.fori_loop(0, SUB // 16, comp_b, cntv)

        cntv = lax.fori_loop(0, NSUB, csc_b, jnp.zeros((16,), jnp.int32))

        c16_v[pl.ds(0, 16)] = cntv
        pltpu.sync_copy(c16_v, counts_sh.at[pl.ds((q * NS + s) * 16, 16)])
        lbase = ((q * NC + c) * NS + s) * CAP
        pltpu.sync_copy(pk_v, pkl_hbm.at[pl.ds(lbase, CAP)])
        pltpu.sync_copy(cf_v, cfl_hbm.at[pl.ds(lbase, CAP)])
        plsc.subcore_barrier()

        def zagg(j, _):
            for u in range(FW // 16):
                agg_v[j, pl.ds(u * 16, 16)] = jnp.zeros((16,), jnp.float32)
            return 0

        lax.fori_loop(0, RH, zagg, 0)

        pltpu.sync_copy(counts_sh.at[pl.ds(q * NS * 16, NS * 16)], cnts_v)

        def lst_b(s2, _):
            cnt2 = cnts_v[pl.ds(s2 * 16, 16)][0]
            sbase = ((q * NC + c) * NS + s2) * CAP

            def win_b(w, _):
                base = sbase + w * WN
                pltpu.sync_copy(pkl_hbm.at[pl.ds(base, WN)], wpk_v)
                pltpu.sync_copy(cfl_hbm.at[pl.ds(base, WN)], wcf_v)
                _zero_1d(lpk_v, WN // 16, jnp.int32)
                _zero_1d(lcf_v, WN // 16, jnp.float32)

                def fil_b(g, lcv):
                    pk16 = wpk_v[pl.ds(g * 16, 16)]
                    cf16 = wcf_v[pl.ds(g * 16, 16)]
                    tq16 = pk16 // PKM
                    src16 = pk16 % PKM
                    mr = (tq16 >= n * RH) & (tq16 < (n + 1) * RH)
                    tql = jnp.clip(tq16 - n * RH, 0, RH - 1)
                    lpk = tql * PKM + src16
                    mi = mr.astype(jnp.int32)
                    ps = _prefix_incl(mi, lane)
                    dest = jnp.where(mr, lcv + ps - mi, WN + lane)
                    plsc.store_scatter(lpk_v, [dest], lpk)
                    plsc.store_scatter(lcf_v, [dest], cf16)
                    return lcv + ps[15]

                lcv = lax.fori_loop(0, WN // 16, fil_b, jnp.zeros((16,),
                                                                  jnp.int32))
                lc = lcv[0]

                def idx_b(g, _):
                    pk16 = lpk_v[pl.ds(g * 16, 16)]
                    widx_v[pl.ds(g * 16, 16)] = (pk16 % PKM) * NF + f
                    return 0

                lax.fori_loop(0, WN // 16, idx_b, 0)

                def sb_b(b2, _):
                    pltpu.async_copy(
                        h3_hbm.at[widx_v.at[pl.ds(b2 * GB, GB)]],
                        rows_v, sem).wait()

                    def grp(g2, _):
                        gg = b2 * (GB // 16) + g2
                        pk16 = lpk_v[pl.ds(gg * 16, 16)]
                        cf16 = lcf_v[pl.ds(gg * 16, 16)]
                        tql16 = pk16 // PKM
                        for j in range(16):
                            tql = tql16[j]
                            cj = cf16[j]
                            r = g2 * 16 + j
                            for u in range(FW // 16):
                                a = agg_v[tql, pl.ds(u * 16, 16)]
                                agg_v[tql, pl.ds(u * 16, 16)] = (
                                    a + cj * rows_v[r, pl.ds(u * 16, 16)])
                        return 0

                    lax.fori_loop(0, GB // 16, grp, 0)
                    return 0

                lax.fori_loop(0, (lc + GB - 1) // GB, sb_b, 0)
                return 0

            lax.fori_loop(0, (cnt2 + WN - 1) // WN, win_b, 0)
            return 0

        lax.fori_loop(0, NS, lst_b, 0)

        def elu_b(j, _):
            for u in range(FW // 16):
                v = agg_v[j, pl.ds(u * 16, 16)]
                agg_v[j, pl.ds(u * 16, 16)] = jnp.where(
                    v > 0.0, v, jnp.exp(v) - 1.0)
            return 0

        lax.fori_loop(0, RH, elu_b, 0)
        pltpu.sync_copy(agg_v,
                        out_hbm.at[pl.ds(((2 * c + q) * NS + s) * RH, RH)])


@jax.jit
def _sc_gat(h3, asrc, adst, src, tgt):
    mesh = plsc.VectorSubcoreMesh(core_axis_name="c", subcore_axis_name="s",
                                  num_cores=NC, num_subcores=NS)
    f32 = jnp.float32
    i32 = jnp.int32
    kern = pl.kernel(
        _sc_body,
        out_type=(
            jax.ShapeDtypeStruct((2 * NC * NS * RH, FW), f32),
            jax.ShapeDtypeStruct((2 * NC * NS * CAP,), i32),
            jax.ShapeDtypeStruct((2 * NC * NS * CAP,), f32),
        ),
        mesh=mesh,
        compiler_params=pltpu.CompilerParams(needs_layout_passes=False),
        scratch_types=[
            pltpu.VMEM((N_NODES,), f32),
            pltpu.VMEM((HP,), f32),
            pltpu.VMEM((HP,), f32),
            pltpu.VMEM((HP,), f32),
            pltpu.VMEM((SUB,), i32),
            pltpu.VMEM((SUB,), i32),
            pltpu.VMEM((CAP,), i32),
            pltpu.VMEM((CAP,), f32),
            pltpu.VMEM((RH, FW), f32),
            pltpu.VMEM((NS * 16,), i32),
            pltpu.VMEM((16,), i32),
            pltpu.VMEM((WN,), i32),
            pltpu.VMEM((WN,), f32),
            pltpu.VMEM((WN + 16,), i32),
            pltpu.VMEM((WN + 16,), f32),
            pltpu.VMEM((WN,), i32),
            pltpu.VMEM((GB, FW), f32),
            pltpu.VMEM((TPH,), f32),
            pltpu.VMEM((TPH,), f32),
            pltpu.VMEM_SHARED((2 * NS * 16,), i32),
            pltpu.VMEM_SHARED((NS * HP,), f32),
            pltpu.VMEM_SHARED((HP,), f32),
            pltpu.SemaphoreType.DMA,
        ],
    )
    return kern(h3, asrc, adst, src, tgt)


def kernel(x, edge_index, W, att_src, att_dst):
    ei = edge_index.astype(jnp.int32)
    src = ei[0]
    tgt = ei[1]
    h, asc, adt = _tc_transform(x, W.T, att_src.reshape(D, 1),
                                att_dst.reshape(D, 1))
    h3 = h.reshape(N_NODES * NF, FW)
    out_p, _, _ = _sc_gat(h3, asc.reshape(-1), adt.reshape(-1), src, tgt)
    o = out_p.reshape(2 * NC, NR, NF, RH, FW).transpose(0, 1, 3, 2, 4)
    return o.reshape(2 * NC, QP, D)[:, :Q, :].reshape(2 * NC * Q, D)


# speedup vs baseline: 1.5373x; 1.5373x over previous
import jax
import jax.numpy as jnp
from jax import lax
from jax.experimental import pallas as pl
from jax.experimental.pallas import tpu as pltpu
from jax.experimental.pallas import tpu_sc as plsc

N_NODES = 10000
D = 256
N_EDGES = 160000

NC = 2
NS = 16
NF = 2
FW = D // NF
NR = NS // NF
HALF = N_NODES // NC
HP = 5120
Q = HALF // 2
QP = 2560
RH = QP // NR
TPH = HP // NS
ECH = N_EDGES // NS
SUB = 2000
NSUB = ECH // SUB
CAP = ECH + 240
TRASH = CAP - 16
WN = 1024
GB = 128
NEG = 0.2
PKM = 16384


def _tc_body(x_ref, wt_ref, as_ref, ad_ref, h_ref, asc_ref, adt_ref):
    h = jnp.dot(x_ref[...], wt_ref[...], preferred_element_type=jnp.float32)
    h_ref[...] = h
    asc_ref[...] = jnp.dot(h, as_ref[...], preferred_element_type=jnp.float32)
    adt_ref[...] = jnp.dot(h, ad_ref[...], preferred_element_type=jnp.float32)


@jax.jit
def _tc_transform(x, wt, att_s, att_d):
    blk = 1000
    grid = N_NODES // blk
    return pl.pallas_call(
        _tc_body,
        grid=(grid,),
        in_specs=[
            pl.BlockSpec((blk, D), lambda i: (i, 0)),
            pl.BlockSpec((D, D), lambda i: (0, 0)),
            pl.BlockSpec((D, 1), lambda i: (0, 0)),
            pl.BlockSpec((D, 1), lambda i: (0, 0)),
        ],
        out_specs=[
            pl.BlockSpec((blk, D), lambda i: (i, 0)),
            pl.BlockSpec((blk, 1), lambda i: (i, 0)),
            pl.BlockSpec((blk, 1), lambda i: (i, 0)),
        ],
        out_shape=[
            jax.ShapeDtypeStruct((N_NODES, D), jnp.float32),
            jax.ShapeDtypeStruct((N_NODES, 1), jnp.float32),
            jax.ShapeDtypeStruct((N_NODES, 1), jnp.float32),
        ],
    )(x, wt, att_s, att_d)


def _zero_1d(ref, n16, dtype):
    z = jnp.zeros((16,), dtype)

    def b(i, _):
        ref[pl.ds(i * 16, 16)] = z
        return 0

    lax.fori_loop(0, n16, b, 0)


def _prefix_incl(mi, lane):
    ps = mi
    for sh in (1, 2, 4, 8):
        pidx = jnp.maximum(lane - sh, 0)
        sh_v = ps.at[pidx].get(mode="promise_in_bounds")
        ps = ps + jnp.where(lane >= sh, sh_v, 0)
    return ps


def _sc_body(h3_hbm, asrc_hbm, adst_hbm, src_hbm, tgt_hbm,
             out_hbm, pkl_hbm, cfl_hbm,
             asrc_v, adh_v, s_v, srcs_v, tgts_v,
             pk_v, cf_v, agg_v, cnts_v, c16_v,
             lpk_v, lcf_v, widx_v, rows_v, acc_v, tmp_v,
             counts_sh, parts_sh, ssum_sh, sem):
    c = lax.axis_index("c")
    s = lax.axis_index("s")
    lo = c * HALF
    f = s % NF
    n = s // NF
    lane = lax.iota(jnp.int32, 16)

    pltpu.sync_copy(asrc_hbm, asrc_v)
    pltpu.sync_copy(adst_hbm.at[pl.ds(lo, HALF)], adh_v.at[pl.ds(0, HALF)])
    _zero_1d(adh_v.at[pl.ds(HALF, HP - HALF)], (HP - HALF) // 16, jnp.float32)
    _zero_1d(s_v, HP // 16, jnp.float32)

    def amax_b(i, acc):
        return jnp.maximum(acc, asrc_v[pl.ds(i * 16, 16)])

    mx = lax.fori_loop(0, N_NODES // 16, amax_b,
                       jnp.full((16,), -jnp.inf, jnp.float32))
    for sh in (8, 4, 2, 1):
        perm = jnp.bitwise_xor(lane, sh)
        mx = jnp.maximum(mx, mx.at[perm].get(mode="promise_in_bounds"))
    astar = mx

    def scan_b(k, _):
        pltpu.sync_copy(src_hbm.at[pl.ds(s * ECH + k * SUB, SUB)], srcs_v)
        pltpu.sync_copy(tgt_hbm.at[pl.ds(s * ECH + k * SUB, SUB)], tgts_v)

        def pa_b(g, _):
            s16 = srcs_v[pl.ds(g * 16, 16)]
            t16 = tgts_v[pl.ds(g * 16, 16)]
            l16 = t16 - lo
            inh = (l16 >= 0) & (l16 < HALF)
            lc = jnp.clip(l16, 0, HALF - 1)
            ad = plsc.load_gather(adh_v, [lc])
            z = plsc.load_gather(asrc_v, [s16]) + ad
            al = jnp.maximum(z, NEG * z)
            zb = astar + ad
            mb = jnp.maximum(jnp.maximum(zb, NEG * zb), 0.0)
            e = jnp.exp(al - mb)
            e = jnp.where(inh, e, 0.0)
            plsc.addupdate_scatter(s_v, [lc], e)
            return 0

        lax.fori_loop(0, SUB // 16, pa_b, 0)
        return 0

    lax.fori_loop(0, NSUB, scan_b, 0)

    pltpu.sync_copy(s_v, parts_sh.at[pl.ds(s * HP, HP)])
    plsc.subcore_barrier()
    _zero_1d(acc_v, TPH // 16, jnp.float32)

    def red_b(k, _):
        pltpu.sync_copy(parts_sh.at[pl.ds(k * HP + s * TPH, TPH)], tmp_v)

        def add_b(i, _):
            a = acc_v[pl.ds(i * 16, 16)]
            acc_v[pl.ds(i * 16, 16)] = a + tmp_v[pl.ds(i * 16, 16)]
            return 0

        lax.fori_loop(0, TPH // 16, add_b, 0)
        return 0

    lax.fori_loop(0, NS, red_b, 0)
    pltpu.sync_copy(acc_v, ssum_sh.at[pl.ds(s * TPH, TPH)])
    plsc.subcore_barrier()
    pltpu.sync_copy(ssum_sh, s_v)

    for q in (0, 1):
        _zero_1d(pk_v, CAP // 16, jnp.int32)
        _zero_1d(cf_v, CAP // 16, jnp.float32)

        def csc_b(k, cntv):
            pltpu.sync_copy(src_hbm.at[pl.ds(s * ECH + k * SUB, SUB)], srcs_v)
            pltpu.sync_copy(tgt_hbm.at[pl.ds(s * ECH + k * SUB, SUB)], tgts_v)

            def comp_b(g, cv):
                s16 = srcs_v[pl.ds(g * 16, 16)]
                t16 = tgts_v[pl.ds(g * 16, 16)]
                l16 = t16 - lo
                mq = (l16 >= q * Q) & (l16 < (q + 1) * Q)
                lc = jnp.clip(l16, 0, HALF - 1)
                ad = plsc.load_gather(adh_v, [lc])
                z = plsc.load_gather(asrc_v, [s16]) + ad
                al = jnp.maximum(z, NEG * z)
                zb = astar + ad
                mb = jnp.maximum(jnp.maximum(zb, NEG * zb), 0.0)
                e = jnp.exp(al - mb)
                ssum = plsc.load_gather(s_v, [lc])
                cf = e / (ssum + 1e-16)
                tq = jnp.clip(l16 - q * Q, 0, QP - 1)
                pk = tq * PKM + s16
                mi = mq.astype(jnp.int32)
                ps = _prefix_incl(mi, lane)
                dest = jnp.where(mq, cv + ps - mi, TRASH + lane)
                plsc.store_scatter(pk_v, [dest], pk)
                plsc.store_scatter(cf_v, [dest], cf)
                return cv + ps[15]

            return lax.fori_loop(0, SUB // 16, comp_b, cntv)

        cntv = lax.fori_loop(0, NSUB, csc_b, jnp.zeros((16,), jnp.int32))

        c16_v[pl.ds(0, 16)] = cntv
        pltpu.sync_copy(c16_v, counts_sh.at[pl.ds((q * NS + s) * 16, 16)])
        lbase = ((q * NC + c) * NS + s) * CAP
        pltpu.sync_copy(pk_v, pkl_hbm.at[pl.ds(lbase, CAP)])
        pltpu.sync_copy(cf_v, cfl_hbm.at[pl.ds(lbase, CAP)])
        plsc.subcore_barrier()

        def zagg(j, _):
            for u in range(FW // 16):
                agg_v[j, pl.ds(u * 16, 16)] = jnp.zeros((16,), jnp.float32)
            return 0

        lax.fori_loop(0, RH, zagg, 0)

        pltpu.sync_copy(counts_sh.at[pl.ds(q * NS * 16, NS * 16)], cnts_v)
        _zero_1d(lpk_v, (WN + 16) // 16, jnp.int32)
        _zero_1d(widx_v, WN // 16, jnp.int32)

        def lst_b(s2, _):
            cnt2 = cnts_v[pl.ds(s2 * 16, 16)][0]
            sbase = ((q * NC + c) * NS + s2) * CAP
            d1 = pltpu.async_copy(pkl_hbm.at[pl.ds(sbase, CAP)], pk_v, sem)
            d2 = pltpu.async_copy(cfl_hbm.at[pl.ds(sbase, CAP)], cf_v, sem)
            d1.wait()
            d2.wait()

            def win_b(w, _):
                w0 = w * WN
                gmax = jnp.minimum(WN // 16, (cnt2 - w0 + 15) // 16)

                def fil_b(g, lcv):
                    base = w0 + g * 16
                    pk16 = pk_v[pl.ds(base, 16)]
                    cf16 = cf_v[pl.ds(base, 16)]
                    tq16 = pk16 // PKM
                    src16 = pk16 % PKM
                    mr = ((tq16 >= n * RH) & (tq16 < (n + 1) * RH)
                          & (base + lane < cnt2))
                    tql = jnp.clip(tq16 - n * RH, 0, RH - 1)
                    lpk = tql * PKM + src16
                    mi = mr.astype(jnp.int32)
                    ps = _prefix_incl(mi, lane)
                    dest = jnp.where(mr, lcv + ps - mi, WN + lane)
                    plsc.store_scatter(lpk_v, [dest], lpk)
                    plsc.store_scatter(lcf_v, [dest], cf16)
                    return lcv + ps[15]

                lcv = lax.fori_loop(0, gmax, fil_b,
                                    jnp.zeros((16,), jnp.int32))
                lc = lcv[0]

                def idx_b(g, _):
                    pk16 = lpk_v[pl.ds(g * 16, 16)]
                    widx_v[pl.ds(g * 16, 16)] = (pk16 % PKM) * NF + f
                    return 0

                lax.fori_loop(0, (lc + 15) // 16, idx_b, 0)

                def sb_b(b2, _):
                    pltpu.async_copy(
                        h3_hbm.at[widx_v.at[pl.ds(b2 * GB, GB)]],
                        rows_v, sem).wait()

                    def grp(g2, _):
                        gg = b2 * (GB // 16) + g2
                        pk16 = lpk_v[pl.ds(gg * 16, 16)]
                        cf16 = lcf_v[pl.ds(gg * 16, 16)]
                        cf16 = jnp.where(gg * 16 + lane < lc, cf16, 0.0)
                        tql16 = pk16 // PKM
                        for j in range(16):
                            tql = tql16[j]
                            cj = cf16[j]
                            r = g2 * 16 + j
                            for u in range(FW // 16):
                                a = agg_v[tql, pl.ds(u * 16, 16)]
                                agg_v[tql, pl.ds(u * 16, 16)] = (
                                    a + cj * rows_v[r, pl.ds(u * 16, 16)])
                        return 0

                    lax.fori_loop(0, GB // 16, grp, 0)
                    return 0

                lax.fori_loop(0, (lc + GB - 1) // GB, sb_b, 0)
                return 0

            lax.fori_loop(0, (cnt2 + WN - 1) // WN, win_b, 0)
            return 0

        lax.fori_loop(0, NS, lst_b, 0)

        def elu_b(j, _):
            for u in range(FW // 16):
                v = agg_v[j, pl.ds(u * 16, 16)]
                agg_v[j, pl.ds(u * 16, 16)] = jnp.where(
                    v > 0.0, v, jnp.exp(v) - 1.0)
            return 0

        lax.fori_loop(0, RH, elu_b, 0)
        pltpu.sync_copy(agg_v,
                        out_hbm.at[pl.ds(((2 * c + q) * NS + s) * RH, RH)])


@jax.jit
def _sc_gat(h3, asrc, adst, src, tgt):
    mesh = plsc.VectorSubcoreMesh(core_axis_name="c", subcore_axis_name="s",
                                  num_cores=NC, num_subcores=NS)
    f32 = jnp.float32
    i32 = jnp.int32
    kern = pl.kernel(
        _sc_body,
        out_type=(
            jax.ShapeDtypeStruct((2 * NC * NS * RH, FW), f32),
            jax.ShapeDtypeStruct((2 * NC * NS * CAP,), i32),
            jax.ShapeDtypeStruct((2 * NC * NS * CAP,), f32),
        ),
        mesh=mesh,
        compiler_params=pltpu.CompilerParams(needs_layout_passes=False),
        scratch_types=[
            pltpu.VMEM((N_NODES,), f32),
            pltpu.VMEM((HP,), f32),
            pltpu.VMEM((HP,), f32),
            pltpu.VMEM((SUB,), i32),
            pltpu.VMEM((SUB,), i32),
            pltpu.VMEM((CAP,), i32),
            pltpu.VMEM((CAP,), f32),
            pltpu.VMEM((RH, FW), f32),
            pltpu.VMEM((NS * 16,), i32),
            pltpu.VMEM((16,), i32),
            pltpu.VMEM((WN + 16,), i32),
            pltpu.VMEM((WN + 16,), f32),
            pltpu.VMEM((WN,), i32),
            pltpu.VMEM((GB, FW), f32),
            pltpu.VMEM((TPH,), f32),
            pltpu.VMEM((TPH,), f32),
            pltpu.VMEM_SHARED((2 * NS * 16,), i32),
            pltpu.VMEM_SHARED((NS * HP,), f32),
            pltpu.VMEM_SHARED((HP,), f32),
            pltpu.SemaphoreType.DMA,
        ],
    )
    return kern(h3, asrc, adst, src, tgt)


def kernel(x, edge_index, W, att_src, att_dst):
    ei = edge_index.astype(jnp.int32)
    src = ei[0]
    tgt = ei[1]
    h, asc, adt = _tc_transform(x, W.T, att_src.reshape(D, 1),
                                att_dst.reshape(D, 1))
    h3 = h.reshape(N_NODES * NF, FW)
    out_p, _, _ = _sc_gat(h3, asc.reshape(-1), adt.reshape(-1), src, tgt)
    o = out_p.reshape(2 * NC, NR, NF, RH, FW).transpose(0, 1, 3, 2, 4)
    return o.reshape(2 * NC, QP, D)[:, :Q, :].reshape(2 * NC * Q, D)


# speedup vs baseline: 5.6101x; 3.6494x over previous
import jax
import jax.numpy as jnp
from jax import lax
from jax.experimental import pallas as pl
from jax.experimental.pallas import tpu as pltpu
from jax.experimental.pallas import tpu_sc as plsc

N_NODES = 10000
D = 256
N_EDGES = 160000

NC = 2
NS = 16
NF = 2
FW = D // NF
NR = NS // NF
HALF = N_NODES // NC
HP = 5120
Q = HALF // 2
QP = 2560
RH = QP // NR
TPH = HP // NS
ECH = N_EDGES // NS
SUB = 2000
NSUB = ECH // SUB
CAP = ECH + 240
TRASH = CAP - 16
GB = 256
NEG = 0.2
PKM = 16384


def _tc_body(x_ref, wt_ref, as_ref, ad_ref, h_ref, asc_ref, adt_ref):
    h = jnp.dot(x_ref[...], wt_ref[...], preferred_element_type=jnp.float32)
    h_ref[...] = h
    asc_ref[...] = jnp.dot(h, as_ref[...], preferred_element_type=jnp.float32)
    adt_ref[...] = jnp.dot(h, ad_ref[...], preferred_element_type=jnp.float32)


@jax.jit
def _tc_transform(x, wt, att_s, att_d):
    blk = 1000
    grid = N_NODES // blk
    return pl.pallas_call(
        _tc_body,
        grid=(grid,),
        in_specs=[
            pl.BlockSpec((blk, D), lambda i: (i, 0)),
            pl.BlockSpec((D, D), lambda i: (0, 0)),
            pl.BlockSpec((D, 1), lambda i: (0, 0)),
            pl.BlockSpec((D, 1), lambda i: (0, 0)),
        ],
        out_specs=[
            pl.BlockSpec((blk, D), lambda i: (i, 0)),
            pl.BlockSpec((blk, 1), lambda i: (i, 0)),
            pl.BlockSpec((blk, 1), lambda i: (i, 0)),
        ],
        out_shape=[
            jax.ShapeDtypeStruct((N_NODES, D), jnp.float32),
            jax.ShapeDtypeStruct((N_NODES, 1), jnp.float32),
            jax.ShapeDtypeStruct((N_NODES, 1), jnp.float32),
        ],
    )(x, wt, att_s, att_d)


def _zero_1d(ref, n16, dtype):
    z = jnp.zeros((16,), dtype)

    def b(i, _):
        ref[pl.ds(i * 16, 16)] = z
        return 0

    lax.fori_loop(0, n16, b, 0)


def _prefix_incl(mi, lane):
    ps = mi
    for sh in (1, 2, 4, 8):
        pidx = jnp.maximum(lane - sh, 0)
        sh_v = ps.at[pidx].get(mode="promise_in_bounds")
        ps = ps + jnp.where(lane >= sh, sh_v, 0)
    return ps


def _sc_body(h3_hbm, asrc_hbm, adst_hbm, src_hbm, tgt_hbm,
             out_hbm, pkl_hbm, cfl_hbm,
             asrc_v, adh_v, s_v, srcs_v, tgts_v,
             pk_v, cf_v, agg_v, cnts_v, c16_v,
             widx_v, rows_v, acc_v, tmp_v,
             counts_sh, parts_sh, ssum_sh, sem):
    c = lax.axis_index("c")
    s = lax.axis_index("s")
    lo = c * HALF
    f = s % NF
    n = s // NF
    lane = lax.iota(jnp.int32, 16)

    pltpu.sync_copy(asrc_hbm, asrc_v)
    pltpu.sync_copy(adst_hbm.at[pl.ds(lo, HALF)], adh_v.at[pl.ds(0, HALF)])
    _zero_1d(adh_v.at[pl.ds(HALF, HP - HALF)], (HP - HALF) // 16, jnp.float32)
    _zero_1d(s_v, HP // 16, jnp.float32)

    def amax_b(i, acc):
        return jnp.maximum(acc, asrc_v[pl.ds(i * 16, 16)])

    mx = lax.fori_loop(0, N_NODES // 16, amax_b,
                       jnp.full((16,), -jnp.inf, jnp.float32))
    for sh in (8, 4, 2, 1):
        perm = jnp.bitwise_xor(lane, sh)
        mx = jnp.maximum(mx, mx.at[perm].get(mode="promise_in_bounds"))
    astar = mx

    def scan_b(k, _):
        pltpu.sync_copy(src_hbm.at[pl.ds(s * ECH + k * SUB, SUB)], srcs_v)
        pltpu.sync_copy(tgt_hbm.at[pl.ds(s * ECH + k * SUB, SUB)], tgts_v)

        def pa_b(g, _):
            s16 = srcs_v[pl.ds(g * 16, 16)]
            t16 = tgts_v[pl.ds(g * 16, 16)]
            l16 = t16 - lo
            inh = (l16 >= 0) & (l16 < HALF)
            lc = jnp.clip(l16, 0, HALF - 1)
            ad = plsc.load_gather(adh_v, [lc])
            z = plsc.load_gather(asrc_v, [s16]) + ad
            al = jnp.maximum(z, NEG * z)
            zb = astar + ad
            mb = jnp.maximum(jnp.maximum(zb, NEG * zb), 0.0)
            e = jnp.exp(al - mb)
            e = jnp.where(inh, e, 0.0)
            plsc.addupdate_scatter(s_v, [lc], e)
            return 0

        lax.fori_loop(0, SUB // 16, pa_b, 0)
        return 0

    lax.fori_loop(0, NSUB, scan_b, 0)

    pltpu.sync_copy(s_v, parts_sh.at[pl.ds(s * HP, HP)])
    plsc.subcore_barrier()
    _zero_1d(acc_v, TPH // 16, jnp.float32)

    def red_b(k, _):
        pltpu.sync_copy(parts_sh.at[pl.ds(k * HP + s * TPH, TPH)], tmp_v)

        def add_b(i, _):
            a = acc_v[pl.ds(i * 16, 16)]
            acc_v[pl.ds(i * 16, 16)] = a + tmp_v[pl.ds(i * 16, 16)]
            return 0

        lax.fori_loop(0, TPH // 16, add_b, 0)
        return 0

    lax.fori_loop(0, NS, red_b, 0)
    pltpu.sync_copy(acc_v, ssum_sh.at[pl.ds(s * TPH, TPH)])
    plsc.subcore_barrier()
    pltpu.sync_copy(ssum_sh, s_v)

    for q in (0, 1):
        _zero_1d(pk_v, CAP // 16, jnp.int32)
        _zero_1d(cf_v, CAP // 16, jnp.float32)

        def csc_b(k, cntv):
            pltpu.sync_copy(src_hbm.at[pl.ds(s * ECH + k * SUB, SUB)], srcs_v)
            pltpu.sync_copy(tgt_hbm.at[pl.ds(s * ECH + k * SUB, SUB)], tgts_v)

            def comp_b(g, cv):
                s16 = srcs_v[pl.ds(g * 16, 16)]
                t16 = tgts_v[pl.ds(g * 16, 16)]
                l16 = t16 - lo
                mq = (l16 >= q * Q) & (l16 < (q + 1) * Q)
                lc = jnp.clip(l16, 0, HALF - 1)
                ad = plsc.load_gather(adh_v, [lc])
                z = plsc.load_gather(asrc_v, [s16]) + ad
                al = jnp.maximum(z, NEG * z)
                zb = astar + ad
                mb = jnp.maximum(jnp.maximum(zb, NEG * zb), 0.0)
                e = jnp.exp(al - mb)
                ssum = plsc.load_gather(s_v, [lc])
                cf = e / (ssum + 1e-16)
                tq = jnp.clip(l16 - q * Q, 0, QP - 1)
                pk = tq * PKM + s16
                mi = mq.astype(jnp.int32)
                ps = _prefix_incl(mi, lane)
                dest = jnp.where(mq, cv + ps - mi, TRASH + lane)
                plsc.store_scatter(pk_v, [dest], pk)
                plsc.store_scatter(cf_v, [dest], cf)
                return cv + ps[15]

            return lax.fori_loop(0, SUB // 16, comp_b, cntv)

        cntv = lax.fori_loop(0, NSUB, csc_b, jnp.zeros((16,), jnp.int32))

        c16_v[pl.ds(0, 16)] = cntv
        pltpu.sync_copy(c16_v, counts_sh.at[pl.ds((q * NS + s) * 16, 16)])
        lbase = ((q * NC + c) * NS + s) * CAP
        pltpu.sync_copy(pk_v, pkl_hbm.at[pl.ds(lbase, CAP)])
        pltpu.sync_copy(cf_v, cfl_hbm.at[pl.ds(lbase, CAP)])
        plsc.subcore_barrier()

        def zagg(j, _):
            for u in range(FW // 16):
                agg_v[j, pl.ds(u * 16, 16)] = jnp.zeros((16,), jnp.float32)
            return 0

        lax.fori_loop(0, RH, zagg, 0)

        pltpu.sync_copy(counts_sh.at[pl.ds(q * NS * 16, NS * 16)], cnts_v)

        def lst_b(s2, _):
            cnt2 = cnts_v[pl.ds(s2 * 16, 16)][0]
            sbase = ((q * NC + c) * NS + s2) * CAP
            d1 = pltpu.async_copy(pkl_hbm.at[pl.ds(sbase, CAP)], pk_v, sem)
            d2 = pltpu.async_copy(cfl_hbm.at[pl.ds(sbase, CAP)], cf_v, sem)
            d1.wait()
            d2.wait()

            def fil_b(g, lcv):
                base = g * 16
                pk16 = pk_v[pl.ds(base, 16)]
                cf16 = cf_v[pl.ds(base, 16)]
                tq16 = pk16 // PKM
                src16 = pk16 % PKM
                mr = ((tq16 >= n * RH) & (tq16 < (n + 1) * RH)
                      & (base + lane < cnt2))
                tql = jnp.clip(tq16 - n * RH, 0, RH - 1)
                lpk = tql * PKM + src16
                mi = mr.astype(jnp.int32)
                ps = _prefix_incl(mi, lane)
                dest = jnp.where(mr, lcv + ps - mi, TRASH + lane)
                plsc.store_scatter(pk_v, [dest], lpk)
                plsc.store_scatter(cf_v, [dest], cf16)
                return lcv + ps[15]

            lcv = lax.fori_loop(0, (cnt2 + 15) // 16, fil_b,
                                jnp.zeros((16,), jnp.int32))
            lc = lcv[0]

            def sb_b(b2, _):
                base = b2 * GB

                def idx_b(g, _):
                    pk16 = pk_v[pl.ds(base + g * 16, 16)]
                    widx_v[pl.ds(g * 16, 16)] = (pk16 % PKM) * NF + f
                    return 0

                lax.fori_loop(0, GB // 16, idx_b, 0)
                pltpu.async_copy(h3_hbm.at[widx_v], rows_v, sem).wait()

                def grp(g2, _):
                    gg = base + g2 * 16
                    pk16 = pk_v[pl.ds(gg, 16)]
                    cf16 = cf_v[pl.ds(gg, 16)]
                    cf16 = jnp.where(gg + lane < lc, cf16, 0.0)
                    tql16 = jnp.clip(pk16 // PKM, 0, RH - 1)
                    for j in range(16):
                        tql = tql16[j]
                        cj = cf16[j]
                        r = g2 * 16 + j
                        for u in range(FW // 16):
                            a = agg_v[tql, pl.ds(u * 16, 16)]
                            agg_v[tql, pl.ds(u * 16, 16)] = (
                                a + cj * rows_v[r, pl.ds(u * 16, 16)])
                    return 0

                lax.fori_loop(0, GB // 16, grp, 0)
                return 0

            lax.fori_loop(0, (lc + GB - 1) // GB, sb_b, 0)
            return 0

        lax.fori_loop(0, NS, lst_b, 0)

        def elu_b(j, _):
            for u in range(FW // 16):
                v = agg_v[j, pl.ds(u * 16, 16)]
                agg_v[j, pl.ds(u * 16, 16)] = jnp.where(
                    v > 0.0, v, jnp.exp(v) - 1.0)
            return 0

        lax.fori_loop(0, RH, elu_b, 0)
        pltpu.sync_copy(agg_v,
                        out_hbm.at[pl.ds(((2 * c + q) * NS + s) * RH, RH)])


@jax.jit
def _sc_gat(h3, asrc, adst, src, tgt):
    mesh = plsc.VectorSubcoreMesh(core_axis_name="c", subcore_axis_name="s",
                                  num_cores=NC, num_subcores=NS)
    f32 = jnp.float32
    i32 = jnp.int32
    kern = pl.kernel(
        _sc_body,
        out_type=(
            jax.ShapeDtypeStruct((2 * NC * NS * RH, FW), f32),
            jax.ShapeDtypeStruct((2 * NC * NS * CAP,), i32),
            jax.ShapeDtypeStruct((2 * NC * NS * CAP,), f32),
        ),
        mesh=mesh,
        compiler_params=pltpu.CompilerParams(needs_layout_passes=False),
        scratch_types=[
            pltpu.VMEM((N_NODES,), f32),
            pltpu.VMEM((HP,), f32),
            pltpu.VMEM((HP,), f32),
            pltpu.VMEM((SUB,), i32),
            pltpu.VMEM((SUB,), i32),
            pltpu.VMEM((CAP,), i32),
            pltpu.VMEM((CAP,), f32),
            pltpu.VMEM((RH, FW), f32),
            pltpu.VMEM((NS * 16,), i32),
            pltpu.VMEM((16,), i32),
            pltpu.VMEM((GB,), i32),
            pltpu.VMEM((GB, FW), f32),
            pltpu.VMEM((TPH,), f32),
            pltpu.VMEM((TPH,), f32),
            pltpu.VMEM_SHARED((2 * NS * 16,), i32),
            pltpu.VMEM_SHARED((NS * HP,), f32),
            pltpu.VMEM_SHARED((HP,), f32),
            pltpu.SemaphoreType.DMA,
        ],
    )
    return kern(h3, asrc, adst, src, tgt)


def kernel(x, edge_index, W, att_src, att_dst):
    ei = edge_index.astype(jnp.int32)
    src = ei[0]
    tgt = ei[1]
    h, asc, adt = _tc_transform(x, W.T, att_src.reshape(D, 1),
                                att_dst.reshape(D, 1))
    h3 = h.reshape(N_NODES * NF, FW)
    out_p, _, _ = _sc_gat(h3, asc.reshape(-1), adt.reshape(-1), src, tgt)
    o = out_p.reshape(2 * NC, NR, NF, RH, FW).transpose(0, 1, 3, 2, 4)
    return o.reshape(2 * NC, QP, D)[:, :Q, :].reshape(2 * NC * Q, D)
